# Initial kernel scaffold; baseline (speedup 1.0000x reference)
import jax
import jax.numpy as jnp
from jax.experimental import pallas as pl


def kernel(inputs, plane_coef):
    raise NotImplementedError("write your pallas kernel here")


# speedup vs baseline: 69.8974x; 69.8974x over previous
import functools

import jax
import jax.numpy as jnp
from jax import lax
from jax.experimental import pallas as pl
from jax.experimental.pallas import tpu as pltpu
from jax.experimental.pallas import tpu_sc as plsc

RES = 512
NCH = 32
NPLANES = 3
PLANE_ROWS = RES * RES
TBL_ROWS = NPLANES * PLANE_ROWS

NC = 2
NS = 16
NW = NC * NS
LANES = 16

CHUNK = 128
NJ = 12


def _transpose_body(x_ref, o_ref):
    o_ref[...] = jnp.transpose(x_ref[...], (0, 2, 1))


_TBLK = 2048


def _build_table(plane_coef):
    x = plane_coef.reshape(NPLANES, NCH, PLANE_ROWS)
    out = pl.pallas_call(
        _transpose_body,
        grid=(NPLANES, PLANE_ROWS // _TBLK),
        in_specs=[pl.BlockSpec((1, NCH, _TBLK), lambda p, i: (p, 0, i))],
        out_specs=pl.BlockSpec((1, _TBLK, NCH), lambda p, i: (p, i, 0)),
        out_shape=jax.ShapeDtypeStruct((NPLANES, PLANE_ROWS, NCH), jnp.float32),
    )(x)
    return out.reshape(TBL_ROWS, NCH)


def _scale_coord(t):
    t = t * (0.5 * (RES - 1)) + (0.5 * (RES - 1))
    t = jnp.minimum(jnp.maximum(t, 0.0), float(RES - 1))
    ti = t.astype(jnp.int32)
    return ti, t - ti.astype(jnp.float32)


def _make_sc_kernel(n_points):
    pts_per_w = n_points // NW
    chunks_per_w = pts_per_w // CHUNK
    ngroups = CHUNK // LANES

    def body(coords_hbm, table_hbm, out_hbm, cbuf, ibuf, wbuf, rbuf, obuf, sem):
        cid = lax.axis_index("c")
        sid = lax.axis_index("s")
        wid = sid * NC + cid
        base_pt = wid * pts_per_w

        def chunk_body(k, carry):
            start = base_pt + k * CHUNK
            pltpu.sync_copy(coords_hbm.at[:, pl.ds(start, CHUNK)], cbuf)

            for g in range(ngroups):
                s = g * LANES
                xi, wx = _scale_coord(cbuf[0, pl.ds(s, LANES)])
                yi, wy = _scale_coord(cbuf[1, pl.ds(s, LANES)])
                zi, wz = _scale_coord(cbuf[2, pl.ds(s, LANES)])
                planes = ((xi, wx, yi, wy), (xi, wx, zi, wz), (yi, wy, zi, wz))
                for p, (ui, wu, vi, wv) in enumerate(planes):
                    v1 = jnp.minimum(vi + 1, RES - 1)
                    r00 = vi * RES + ui + p * PLANE_ROWS
                    r10 = v1 * RES + ui + p * PLANE_ROWS
                    r01 = jnp.minimum(r00 + 1, TBL_ROWS - 1)
                    r11 = jnp.minimum(r10 + 1, TBL_ROWS - 1)
                    ou = 1.0 - wu
                    ov = 1.0 - wv
                    j0 = p * 4
                    ibuf[j0 + 0, pl.ds(s, LANES)] = r00
                    ibuf[j0 + 1, pl.ds(s, LANES)] = r01
                    ibuf[j0 + 2, pl.ds(s, LANES)] = r10
                    ibuf[j0 + 3, pl.ds(s, LANES)] = r11
                    wbuf[pl.ds((j0 + 0) * CHUNK + s, LANES)] = ou * ov
                    wbuf[pl.ds((j0 + 1) * CHUNK + s, LANES)] = wu * ov
                    wbuf[pl.ds((j0 + 2) * CHUNK + s, LANES)] = ou * wv
                    wbuf[pl.ds((j0 + 3) * CHUNK + s, LANES)] = wu * wv

            copies = [
                pltpu.async_copy(table_hbm.at[ibuf.at[j]],
                                 rbuf.at[pl.ds(j * CHUNK, CHUNK)], sem)
                for j in range(NJ)
            ]
            for cp in copies:
                cp.wait()

            def acc_group(g, c2):
                s = g * LANES
                wvs = [wbuf[pl.ds(j * CHUNK + s, LANES)] for j in range(NJ)]
                for l in range(LANES):
                    b = s + l
                    a0 = jnp.zeros((LANES,), jnp.float32)
                    a1 = jnp.zeros((LANES,), jnp.float32)
                    for j in range(NJ):
                        w = wvs[j][l]
                        r = j * CHUNK + b
                        a0 = a0 + w * rbuf[r, pl.ds(0, LANES)]
                        a1 = a1 + w * rbuf[r, pl.ds(LANES, LANES)]
                    obuf[b, pl.ds(0, LANES)] = a0
                    obuf[b, pl.ds(LANES, LANES)] = a1
                return c2

            lax.fori_loop(0, ngroups, acc_group, 0)
            pltpu.sync_copy(obuf, out_hbm.at[pl.ds(start, CHUNK), :])
            return carry

        lax.fori_loop(0, chunks_per_w, chunk_body, 0)

    return pl.kernel(
        body,
        out_type=jax.ShapeDtypeStruct((n_points, NCH), jnp.float32),
        mesh=plsc.VectorSubcoreMesh(core_axis_name="c", subcore_axis_name="s",
                                    num_cores=NC, num_subcores=NS),
        compiler_params=pltpu.CompilerParams(use_tc_tiling_on_sc=False),
        scratch_types=[
            pltpu.VMEM((3, CHUNK), jnp.float32),
            pltpu.VMEM((NJ, CHUNK), jnp.int32),
            pltpu.VMEM((NJ * CHUNK,), jnp.float32),
            pltpu.VMEM((NJ * CHUNK, NCH), jnp.float32),
            pltpu.VMEM((CHUNK, NCH), jnp.float32),
            pltpu.SemaphoreType.DMA,
        ],
    )


@jax.jit
def _triplane(inputs, plane_coef):
    n = inputs.shape[0]
    tbl = _build_table(plane_coef)
    coords_t = inputs.T
    return _make_sc_kernel(n)(coords_t, tbl)


def kernel(inputs, plane_coef):
    original_shape = inputs.shape[:-1]
    flat = inputs.reshape(-1, 3)
    out = _triplane(flat, plane_coef)
    return out.reshape(*original_shape, NCH)


# speedup vs baseline: 87.5665x; 1.2528x over previous
import functools

import jax
import jax.numpy as jnp
from jax import lax
from jax.experimental import pallas as pl
from jax.experimental.pallas import tpu as pltpu
from jax.experimental.pallas import tpu_sc as plsc

RES = 512
NCH = 32
NPLANES = 3
PLANE_ROWS = RES * RES
TBL_ROWS = NPLANES * PLANE_ROWS

NC = 2
NS = 16
NW = NC * NS
LANES = 16

CHUNK = 128
NJ = 12


def _transpose_body(x_ref, o_ref):
    o_ref[...] = jnp.transpose(x_ref[...], (0, 2, 1))


_TBLK = 2048


def _build_table(plane_coef):
    x = plane_coef.reshape(NPLANES, NCH, PLANE_ROWS)
    out = pl.pallas_call(
        _transpose_body,
        grid=(NPLANES, PLANE_ROWS // _TBLK),
        in_specs=[pl.BlockSpec((1, NCH, _TBLK), lambda p, i: (p, 0, i))],
        out_specs=pl.BlockSpec((1, _TBLK, NCH), lambda p, i: (p, i, 0)),
        out_shape=jax.ShapeDtypeStruct((NPLANES, PLANE_ROWS, NCH), jnp.float32),
    )(x)
    return out.reshape(TBL_ROWS, NCH)


def _scale_coord(t):
    t = t * (0.5 * (RES - 1)) + (0.5 * (RES - 1))
    t = jnp.minimum(jnp.maximum(t, 0.0), float(RES - 1))
    ti = t.astype(jnp.int32)
    return ti, t - ti.astype(jnp.float32)


def _make_sc_kernel(n_points):
    pts_per_w = n_points // NW
    chunks_per_w = pts_per_w // CHUNK
    ngroups = CHUNK // LANES

    def body(coords_hbm, table_hbm, out_hbm, cbuf, ibuf, wbuf, rbuf, obuf,
             sem0, sem1):
        cid = lax.axis_index("c")
        sid = lax.axis_index("s")
        wid = sid * NC + cid
        base_pt = wid * pts_per_w
        sems = (sem0, sem1)

        def prep(slot, start):
            cb = cbuf.at[slot]
            ib = ibuf.at[slot]
            wb = wbuf.at[slot]
            pltpu.sync_copy(coords_hbm.at[:, pl.ds(start, CHUNK)], cb)
            for g in range(ngroups):
                s = g * LANES
                xi, wx = _scale_coord(cb[0, pl.ds(s, LANES)])
                yi, wy = _scale_coord(cb[1, pl.ds(s, LANES)])
                zi, wz = _scale_coord(cb[2, pl.ds(s, LANES)])
                planes = ((xi, wx, yi, wy), (xi, wx, zi, wz), (yi, wy, zi, wz))
                for p, (ui, wu, vi, wv) in enumerate(planes):
                    v1 = jnp.minimum(vi + 1, RES - 1)
                    r00 = vi * RES + ui + p * PLANE_ROWS
                    r10 = v1 * RES + ui + p * PLANE_ROWS
                    r01 = jnp.minimum(r00 + 1, TBL_ROWS - 1)
                    r11 = jnp.minimum(r10 + 1, TBL_ROWS - 1)
                    ou = 1.0 - wu
                    ov = 1.0 - wv
                    j0 = p * 4
                    ib[j0 + 0, pl.ds(s, LANES)] = r00
                    ib[j0 + 1, pl.ds(s, LANES)] = r01
                    ib[j0 + 2, pl.ds(s, LANES)] = r10
                    ib[j0 + 3, pl.ds(s, LANES)] = r11
                    wb[pl.ds((j0 + 0) * CHUNK + s, LANES)] = ou * ov
                    wb[pl.ds((j0 + 1) * CHUNK + s, LANES)] = wu * ov
                    wb[pl.ds((j0 + 2) * CHUNK + s, LANES)] = ou * wv
                    wb[pl.ds((j0 + 3) * CHUNK + s, LANES)] = wu * wv
            for j in range(NJ):
                pltpu.async_copy(table_hbm.at[ibuf.at[slot].at[j]],
                                 rbuf.at[slot].at[pl.ds(j * CHUNK, CHUNK)],
                                 sems[slot])

        def drain(slot):
            for j in range(NJ):
                pltpu.make_async_copy(
                    table_hbm.at[ibuf.at[slot].at[j]],
                    rbuf.at[slot].at[pl.ds(j * CHUNK, CHUNK)],
                    sems[slot]).wait()

        def accum(slot, start):
            rb = rbuf.at[slot]
            wb = wbuf.at[slot]
            ob = obuf.at[slot]

            def acc_group(g, c2):
                s = g * LANES
                wvs = [wb[pl.ds(j * CHUNK + s, LANES)] for j in range(NJ)]
                for l in range(LANES):
                    b = s + l
                    a0 = jnp.zeros((LANES,), jnp.float32)
                    a1 = jnp.zeros((LANES,), jnp.float32)
                    for j in range(NJ):
                        w = wvs[j][l]
                        r = j * CHUNK + b
                        a0 = a0 + w * rb[r, pl.ds(0, LANES)]
                        a1 = a1 + w * rb[r, pl.ds(LANES, LANES)]
                    ob[b, pl.ds(0, LANES)] = a0
                    ob[b, pl.ds(LANES, LANES)] = a1
                return c2

            lax.fori_loop(0, ngroups, acc_group, 0)
            pltpu.sync_copy(ob, out_hbm.at[pl.ds(start, CHUNK), :])

        npairs = chunks_per_w // 2

        def pair_body(i, carry):
            k0 = base_pt + (2 * i) * CHUNK
            k1 = k0 + CHUNK
            prep(1, k1)
            drain(0)
            accum(0, k0)

            @pl.when(i < npairs - 1)
            def _():
                prep(0, k1 + CHUNK)

            drain(1)
            accum(1, k1)
            return carry

        prep(0, base_pt)
        lax.fori_loop(0, npairs, pair_body, 0)

    return pl.kernel(
        body,
        out_type=jax.ShapeDtypeStruct((n_points, NCH), jnp.float32),
        mesh=plsc.VectorSubcoreMesh(core_axis_name="c", subcore_axis_name="s",
                                    num_cores=NC, num_subcores=NS),
        compiler_params=pltpu.CompilerParams(use_tc_tiling_on_sc=False),
        scratch_types=[
            pltpu.VMEM((2, 3, CHUNK), jnp.float32),
            pltpu.VMEM((2, NJ, CHUNK), jnp.int32),
            pltpu.VMEM((2, NJ * CHUNK), jnp.float32),
            pltpu.VMEM((2, NJ * CHUNK, NCH), jnp.float32),
            pltpu.VMEM((2, CHUNK, NCH), jnp.float32),
            pltpu.SemaphoreType.DMA,
            pltpu.SemaphoreType.DMA,
        ],
    )


@jax.jit
def _triplane(inputs, plane_coef):
    n = inputs.shape[0]
    tbl = _build_table(plane_coef)
    coords_t = inputs.T
    return _make_sc_kernel(n)(coords_t, tbl)


def kernel(inputs, plane_coef):
    original_shape = inputs.shape[:-1]
    flat = inputs.reshape(-1, 3)
    out = _triplane(flat, plane_coef)
    return out.reshape(*original_shape, NCH)


# speedup vs baseline: 112.4966x; 1.2847x over previous
import functools

import jax
import jax.numpy as jnp
from jax import lax
from jax.experimental import pallas as pl
from jax.experimental.pallas import tpu as pltpu
from jax.experimental.pallas import tpu_sc as plsc

RES = 512
NCH = 32
NPLANES = 3
PLANE_ROWS = RES * RES
TBL_ROWS = NPLANES * PLANE_ROWS

NC = 2
NS = 16
NW = NC * NS
LANES = 16

CHUNK = 128
NJ = 12


_BV = 8


def _pack_body(x_ref, o_ref):
    x = x_ref[0]
    parts = [jnp.transpose(x[:, m, :], (1, 0)) for m in range(_BV)]
    grp0 = jnp.concatenate(parts[0:4], axis=1)
    grp1 = jnp.concatenate(parts[4:8], axis=1)
    o_ref[...] = jnp.concatenate([grp0, grp1], axis=0)


def _build_table(plane_coef):
    out = pl.pallas_call(
        _pack_body,
        grid=(NPLANES, RES // _BV),
        in_specs=[pl.BlockSpec((1, NCH, _BV, RES), lambda p, i: (p, 0, i, 0))],
        out_specs=pl.BlockSpec((2 * RES, 128),
                               lambda p, i: (p * (RES // _BV) + i, 0)),
        out_shape=jax.ShapeDtypeStruct((TBL_ROWS * NCH // 128, 128),
                                       jnp.float32),
    )(plane_coef)
    return out.reshape(TBL_ROWS, NCH)


def _scale_coord(t):
    t = t * (0.5 * (RES - 1)) + (0.5 * (RES - 1))
    t = jnp.minimum(jnp.maximum(t, 0.0), float(RES - 1))
    ti = t.astype(jnp.int32)
    return ti, t - ti.astype(jnp.float32)


def _make_sc_kernel(n_points):
    pts_per_w = n_points // NW
    chunks_per_w = pts_per_w // CHUNK
    ngroups = CHUNK // LANES

    def body(coords_hbm, table_hbm, out_hbm, cbuf, ibuf, wbuf, rbuf, obuf,
             sem0, sem1):
        tbl = table_hbm
        cid = lax.axis_index("c")
        sid = lax.axis_index("s")
        wid = sid * NC + cid
        base_pt = wid * pts_per_w
        sems = (sem0, sem1)

        def prep(slot, start):
            cb = cbuf.at[slot]
            ib = ibuf.at[slot]
            wb = wbuf.at[slot]
            pltpu.sync_copy(coords_hbm.at[:, pl.ds(start, CHUNK)], cb)
            for g in range(ngroups):
                s = g * LANES
                xi, wx = _scale_coord(cb[0, pl.ds(s, LANES)])
                yi, wy = _scale_coord(cb[1, pl.ds(s, LANES)])
                zi, wz = _scale_coord(cb[2, pl.ds(s, LANES)])
                planes = ((xi, wx, yi, wy), (xi, wx, zi, wz), (yi, wy, zi, wz))
                for p, (ui, wu, vi, wv) in enumerate(planes):
                    v1 = jnp.minimum(vi + 1, RES - 1)
                    u4 = ui * 4 + p * PLANE_ROWS
                    r00 = ((vi & ~3) * RES + (vi & 3)) + u4
                    r10 = ((v1 & ~3) * RES + (v1 & 3)) + u4
                    r01 = jnp.minimum(r00 + 4, TBL_ROWS - 1)
                    r11 = jnp.minimum(r10 + 4, TBL_ROWS - 1)
                    ou = 1.0 - wu
                    ov = 1.0 - wv
                    j0 = p * 4
                    ib[j0 + 0, pl.ds(s, LANES)] = r00
                    ib[j0 + 1, pl.ds(s, LANES)] = r01
                    ib[j0 + 2, pl.ds(s, LANES)] = r10
                    ib[j0 + 3, pl.ds(s, LANES)] = r11
                    wb[pl.ds((j0 + 0) * CHUNK + s, LANES)] = ou * ov
                    wb[pl.ds((j0 + 1) * CHUNK + s, LANES)] = wu * ov
                    wb[pl.ds((j0 + 2) * CHUNK + s, LANES)] = ou * wv
                    wb[pl.ds((j0 + 3) * CHUNK + s, LANES)] = wu * wv
            for j in range(NJ):
                pltpu.async_copy(tbl.at[ibuf.at[slot].at[j]],
                                 rbuf.at[slot].at[pl.ds(j * CHUNK, CHUNK)],
                                 sems[slot])

        def drain(slot):
            for j in range(NJ):
                pltpu.make_async_copy(
                    tbl.at[ibuf.at[slot].at[j]],
                    rbuf.at[slot].at[pl.ds(j * CHUNK, CHUNK)],
                    sems[slot]).wait()

        def accum(slot, start):
            rb = rbuf.at[slot]
            wb = wbuf.at[slot]
            ob = obuf.at[slot]

            def acc_group(g, c2):
                s = g * LANES
                wvs = [wb[pl.ds(j * CHUNK + s, LANES)] for j in range(NJ)]
                for l in range(LANES):
                    b = s + l
                    a0 = jnp.zeros((LANES,), jnp.float32)
                    a1 = jnp.zeros((LANES,), jnp.float32)
                    for j in range(NJ):
                        w = wvs[j][l]
                        r = j * CHUNK + b
                        a0 = a0 + w * rb[r, pl.ds(0, LANES)]
                        a1 = a1 + w * rb[r, pl.ds(LANES, LANES)]
                    ob[pl.ds(b * NCH, LANES)] = a0
                    ob[pl.ds(b * NCH + LANES, LANES)] = a1
                return c2

            lax.fori_loop(0, ngroups, acc_group, 0)
            pltpu.sync_copy(ob, out_hbm.at[pl.ds(start * NCH, CHUNK * NCH)])

        npairs = chunks_per_w // 2

        def pair_body(i, carry):
            k0 = base_pt + (2 * i) * CHUNK
            k1 = k0 + CHUNK
            prep(1, k1)
            drain(0)
            accum(0, k0)

            @pl.when(i < npairs - 1)
            def _():
                prep(0, k1 + CHUNK)

            drain(1)
            accum(1, k1)
            return carry

        prep(0, base_pt)
        lax.fori_loop(0, npairs, pair_body, 0)

    return pl.kernel(
        body,
        out_type=jax.ShapeDtypeStruct((n_points * NCH,), jnp.float32),
        mesh=plsc.VectorSubcoreMesh(core_axis_name="c", subcore_axis_name="s",
                                    num_cores=NC, num_subcores=NS),
        compiler_params=pltpu.CompilerParams(use_tc_tiling_on_sc=False),
        scratch_types=[
            pltpu.VMEM((2, 3, CHUNK), jnp.float32),
            pltpu.VMEM((2, NJ, CHUNK), jnp.int32),
            pltpu.VMEM((2, NJ * CHUNK), jnp.float32),
            pltpu.VMEM((2, NJ * CHUNK, NCH), jnp.float32),
            pltpu.VMEM((2, CHUNK * NCH), jnp.float32),
            pltpu.SemaphoreType.DMA,
            pltpu.SemaphoreType.DMA,
        ],
    )


@jax.jit
def _triplane(inputs, plane_coef):
    n = inputs.shape[0]
    tbl = _build_table(plane_coef)
    coords_t = inputs.T
    return _make_sc_kernel(n)(coords_t, tbl)


def kernel(inputs, plane_coef):
    original_shape = inputs.shape[:-1]
    flat = inputs.reshape(-1, 3)
    out = _triplane(flat, plane_coef)
    return out.reshape(*original_shape, NCH)


# speedup vs baseline: 112.6309x; 1.0012x over previous
import functools

import jax
import jax.numpy as jnp
from jax import lax
from jax.experimental import pallas as pl
from jax.experimental.pallas import tpu as pltpu
from jax.experimental.pallas import tpu_sc as plsc

RES = 512
NCH = 32
NPLANES = 3
PLANE_ROWS = RES * RES
TBL_ROWS = NPLANES * PLANE_ROWS

NC = 2
NS = 16
NW = NC * NS
LANES = 16

CHUNK = 128
NJ = 12


_BV = 8


def _pack_body(x_ref, o_ref):
    x = x_ref[0]
    parts = [jnp.transpose(x[:, m, :], (1, 0)) for m in range(_BV)]
    grp0 = jnp.concatenate(parts[0:4], axis=1)
    grp1 = jnp.concatenate(parts[4:8], axis=1)
    o_ref[...] = jnp.concatenate([grp0, grp1], axis=0)


def _build_table(plane_coef):
    out = pl.pallas_call(
        _pack_body,
        grid=(NPLANES, RES // _BV),
        in_specs=[pl.BlockSpec((1, NCH, _BV, RES), lambda p, i: (p, 0, i, 0))],
        out_specs=pl.BlockSpec((2 * RES, 128),
                               lambda p, i: (p * (RES // _BV) + i, 0)),
        out_shape=jax.ShapeDtypeStruct((TBL_ROWS * NCH // 128, 128),
                                       jnp.float32),
    )(plane_coef)
    return out.reshape(TBL_ROWS, NCH)


def _scale_coord(t):
    t = t * (0.5 * (RES - 1)) + (0.5 * (RES - 1))
    t = jnp.minimum(jnp.maximum(t, 0.0), float(RES - 1))
    ti = t.astype(jnp.int32)
    return ti, t - ti.astype(jnp.float32)


_RPC = CHUNK * NCH // 128


def _make_sc_kernel(n_points):
    pts_per_w = n_points // NW
    chunks_per_w = pts_per_w // CHUNK
    ngroups = CHUNK // LANES
    rows_per_w = pts_per_w * NCH // 128

    def body(coords_hbm, table_hbm, out_hbm, cbuf, ibuf, wbuf, rbuf, obuf,
             sem0, sem1):
        tbl = table_hbm
        cid = lax.axis_index("c")
        sid = lax.axis_index("s")
        wid = sid * NC + cid
        base_pt = wid * pts_per_w
        base_row = wid * rows_per_w
        sems = (sem0, sem1)

        def prep(slot, start):
            cb = cbuf.at[slot]
            ib = ibuf.at[slot]
            wb = wbuf.at[slot]
            pltpu.sync_copy(coords_hbm.at[:, pl.ds(start, CHUNK)], cb)
            for g in range(ngroups):
                s = g * LANES
                xi, wx = _scale_coord(cb[0, pl.ds(s, LANES)])
                yi, wy = _scale_coord(cb[1, pl.ds(s, LANES)])
                zi, wz = _scale_coord(cb[2, pl.ds(s, LANES)])
                planes = ((xi, wx, yi, wy), (xi, wx, zi, wz), (yi, wy, zi, wz))
                for p, (ui, wu, vi, wv) in enumerate(planes):
                    v1 = jnp.minimum(vi + 1, RES - 1)
                    u4 = ui * 4 + p * PLANE_ROWS
                    r00 = ((vi & ~3) * RES + (vi & 3)) + u4
                    r10 = ((v1 & ~3) * RES + (v1 & 3)) + u4
                    r01 = jnp.minimum(r00 + 4, TBL_ROWS - 1)
                    r11 = jnp.minimum(r10 + 4, TBL_ROWS - 1)
                    ou = 1.0 - wu
                    ov = 1.0 - wv
                    j0 = p * 4
                    ib[j0 + 0, pl.ds(s, LANES)] = r00
                    ib[j0 + 1, pl.ds(s, LANES)] = r01
                    ib[j0 + 2, pl.ds(s, LANES)] = r10
                    ib[j0 + 3, pl.ds(s, LANES)] = r11
                    wb[pl.ds((j0 + 0) * CHUNK + s, LANES)] = ou * ov
                    wb[pl.ds((j0 + 1) * CHUNK + s, LANES)] = wu * ov
                    wb[pl.ds((j0 + 2) * CHUNK + s, LANES)] = ou * wv
                    wb[pl.ds((j0 + 3) * CHUNK + s, LANES)] = wu * wv
            for j in range(NJ):
                pltpu.async_copy(tbl.at[ibuf.at[slot].at[j]],
                                 rbuf.at[slot].at[pl.ds(j * CHUNK, CHUNK)],
                                 sems[slot])

        def drain(slot):
            for j in range(NJ):
                pltpu.make_async_copy(
                    tbl.at[ibuf.at[slot].at[j]],
                    rbuf.at[slot].at[pl.ds(j * CHUNK, CHUNK)],
                    sems[slot]).wait()

        def accum(slot, row0):
            rb = rbuf.at[slot]
            wb = wbuf.at[slot]
            ob = obuf.at[slot]

            def acc_group(g, c2):
                s = g * LANES
                wvs = [wb[pl.ds(j * CHUNK + s, LANES)] for j in range(NJ)]
                for l in range(LANES):
                    b = s + l
                    a0 = jnp.zeros((LANES,), jnp.float32)
                    a1 = jnp.zeros((LANES,), jnp.float32)
                    for j in range(NJ):
                        w = wvs[j][l]
                        r = j * CHUNK + b
                        a0 = a0 + w * rb[r, pl.ds(0, LANES)]
                        a1 = a1 + w * rb[r, pl.ds(LANES, LANES)]
                    row = g * 4 + (l // 4)
                    col = (l % 4) * NCH
                    ob[row, pl.ds(col, LANES)] = a0
                    ob[row, pl.ds(col + LANES, LANES)] = a1
                return c2

            lax.fori_loop(0, ngroups, acc_group, 0)
            pltpu.sync_copy(ob, out_hbm.at[pl.ds(row0, _RPC), :])

        npairs = chunks_per_w // 2

        def pair_body(i, carry):
            k0 = base_pt + (2 * i) * CHUNK
            k1 = k0 + CHUNK
            r0 = base_row + (2 * i) * _RPC
            prep(1, k1)
            drain(0)
            accum(0, r0)

            @pl.when(i < npairs - 1)
            def _():
                prep(0, k1 + CHUNK)

            drain(1)
            accum(1, r0 + _RPC)
            return carry

        prep(0, base_pt)
        lax.fori_loop(0, npairs, pair_body, 0)

    return pl.kernel(
        body,
        out_type=jax.ShapeDtypeStruct((n_points * NCH // 128, 128),
                                      jnp.float32),
        mesh=plsc.VectorSubcoreMesh(core_axis_name="c", subcore_axis_name="s",
                                    num_cores=NC, num_subcores=NS),
        compiler_params=pltpu.CompilerParams(use_tc_tiling_on_sc=False),
        scratch_types=[
            pltpu.VMEM((2, 3, CHUNK), jnp.float32),
            pltpu.VMEM((2, NJ, CHUNK), jnp.int32),
            pltpu.VMEM((2, NJ * CHUNK), jnp.float32),
            pltpu.VMEM((2, NJ * CHUNK, NCH), jnp.float32),
            pltpu.VMEM((2, _RPC, 128), jnp.float32),
            pltpu.SemaphoreType.DMA,
            pltpu.SemaphoreType.DMA,
        ],
    )


@jax.jit
def _triplane(inputs, plane_coef):
    n = inputs.shape[0]
    tbl = _build_table(plane_coef)
    coords_t = inputs.T
    return _make_sc_kernel(n)(coords_t, tbl)


def kernel(inputs, plane_coef):
    original_shape = inputs.shape[:-1]
    flat = inputs.reshape(-1, 3)
    out = _triplane(flat, plane_coef)
    return out.reshape(*original_shape, NCH)
